# baseline (speedup 1.0000x reference)
import functools

import jax
import jax.numpy as jnp
from jax import lax
from jax.experimental import pallas as pl
from jax.experimental.pallas import tpu as pltpu
from jax.experimental.pallas import tpu_sc as plsc

_L = 200
_D = 64
_B = 1024
_LANES = 16
_NVEC = _D // _LANES
_BT = _B // 128
_NSLAB = _L * _BT
_PAD = 129


@functools.lru_cache(maxsize=None)
def _build(vocab: int):
    info = plsc.get_sparse_core_info()
    nw = info.num_cores * info.num_subcores
    per_w = _NSLAB // nw
    half = per_w // 2

    mesh = plsc.VectorSubcoreMesh(core_axis_name="c", subcore_axis_name="s")

    @functools.partial(
        pl.kernel,
        mesh=mesh,
        compiler_params=pltpu.CompilerParams(
            use_tc_tiling_on_sc=False, needs_layout_passes=False),
        name="tok_pos_embed",
        out_type=jax.ShapeDtypeStruct((_L, _D // 8, _BT, 8, 128), jnp.float32),
        scratch_types=[
            pltpu.VMEM((per_w, 128), jnp.int32),
            pltpu.VMEM((4, 128, _D), jnp.float32),
            pltpu.VMEM((4, _D // 8, 8, _PAD), jnp.float32),
            pltpu.VMEM((_L, _D), jnp.float32),
            pltpu.SemaphoreType.DMA,
            pltpu.SemaphoreType.DMA,
            pltpu.SemaphoreType.DMA,
            pltpu.SemaphoreType.DMA,
            pltpu.SemaphoreType.DMA,
            pltpu.SemaphoreType.DMA,
            pltpu.SemaphoreType.DMA,
            pltpu.SemaphoreType.DMA,
            pltpu.SemaphoreType.DMA,
        ],
    )
    def k(x4_hbm, tok_hbm, pos_hbm, out_hbm, idx_all, rows_v, pout, pos_v,
          gsem0, gsem1, gsem2, gsem3, ssem0, ssem1, ssem2, ssem3, isem):
        gsem = (gsem0, gsem1, gsem2, gsem3)
        ssem = (ssem0, ssem1, ssem2, ssem3)
        wid = lax.axis_index("s") * info.num_cores + lax.axis_index("c")
        first = wid * per_w

        idescs = []
        for sl in range(per_w):
            s = first + sl
            l = s >> 3
            idescs.append(pltpu.make_async_copy(
                x4_hbm.at[l >> 3, s & 7, l & 7], idx_all.at[sl], isem))
        for d in idescs:
            d.start()
        pltpu.sync_copy(pos_hbm, pos_v)
        for d in idescs:
            d.wait()

        io = lax.iota(jnp.int32, 16)
        dt_idx = [(16 * kk + io) >> 3 for kk in range(_NVEC)]
        di_idx = [(16 * kk + io) & 7 for kk in range(_NVEC)]
        col_idx = [16 * kk + io for kk in range(_NVEC)]

        def gather_desc(buf, sl):
            return pltpu.make_async_copy(
                tok_hbm.at[idx_all.at[sl]], rows_v.at[buf], gsem[buf])

        def out_desc(buf, s):
            l = s >> 3
            bt = s & 7
            return pltpu.make_async_copy(
                pout.at[buf, :, :, pl.ds(0, 128)],
                out_hbm.at[l, :, bt], ssem[buf])

        def compute(buf, s):
            l = s >> 3
            pv = [pos_v[l, pl.ds(16 * kk, 16)] for kk in range(_NVEC)]

            @plsc.parallel_loop(0, 16, unroll=2)
            def _(r):
                rot = (io + r) & 15
                for m in range(8):
                    bvec = rot + (m * 16)
                    for kk in range(_NVEC):
                        v = plsc.load_gather(
                            rows_v.at[buf], [bvec, col_idx[kk]])
                        v = v + pv[kk]
                        plsc.store_scatter(
                            pout.at[buf], [dt_idx[kk], di_idx[kk], bvec], v)

        for i in range(4):
            gather_desc(i, i).start()

        n_main = (per_w - 2) // 4

        def gbody(h, carry):
            for buf in range(4):
                sl = 4 * h + buf
                s = first + sl
                gather_desc(buf, sl).wait()

                @pl.when(h > 0)
                def _():
                    out_desc(buf, s).wait()

                compute(buf, s)
                out_desc(buf, s).start()

                @pl.when(sl + 4 < per_w)
                def _():
                    gather_desc(buf, sl + 4).start()
            return carry

        lax.fori_loop(0, n_main, gbody, 0)
        for buf in range(2):
            sl = 4 * n_main + buf
            s = first + sl
            gather_desc(buf, sl).wait()
            out_desc(buf, s).wait()
            compute(buf, s)
            out_desc(buf, s).start()
        out_desc(0, first + per_w - 2).wait()
        out_desc(1, first + per_w - 1).wait()
        out_desc(2, first + per_w - 4).wait()
        out_desc(3, first + per_w - 3).wait()

    return k


def kernel(x, token_table, pos_table):
    b, l = x.shape
    x4 = x.astype(jnp.int32).reshape(8, 128, l // 8, 8).transpose(2, 0, 3, 1)
    p = _build(token_table.shape[0])(x4, token_table, pos_table)
    return p.transpose(0, 1, 3, 2, 4).reshape(l, _D, b).transpose(2, 0, 1)


# speedup vs baseline: 1.0488x; 1.0488x over previous
import functools

import jax
import jax.numpy as jnp
from jax import lax
from jax.experimental import pallas as pl
from jax.experimental.pallas import tpu as pltpu
from jax.experimental.pallas import tpu_sc as plsc

_L = 200
_D = 64
_B = 1024
_LANES = 16
_NVEC = _D // _LANES
_BT = _B // 128
_NSLAB = _L * _BT
_PAD = 129


@functools.lru_cache(maxsize=None)
def _build(vocab: int):
    info = plsc.get_sparse_core_info()
    nw = info.num_cores * info.num_subcores
    per_w = _NSLAB // nw
    half = per_w // 2

    mesh = plsc.VectorSubcoreMesh(core_axis_name="c", subcore_axis_name="s")

    @functools.partial(
        pl.kernel,
        mesh=mesh,
        compiler_params=pltpu.CompilerParams(
            use_tc_tiling_on_sc=False, needs_layout_passes=False),
        name="tok_pos_embed",
        out_type=jax.ShapeDtypeStruct((_L, _D // 8, _BT, 8, 128), jnp.float32),
        scratch_types=[
            pltpu.VMEM((per_w, 128), jnp.int32),
            pltpu.VMEM((4, 128, 128), jnp.float32),
            pltpu.VMEM((4, _D // 8, 8, _PAD), jnp.float32),
            pltpu.VMEM((8, 128), jnp.float32),
            pltpu.SemaphoreType.DMA,
            pltpu.SemaphoreType.DMA,
            pltpu.SemaphoreType.DMA,
            pltpu.SemaphoreType.DMA,
            pltpu.SemaphoreType.DMA,
            pltpu.SemaphoreType.DMA,
            pltpu.SemaphoreType.DMA,
            pltpu.SemaphoreType.DMA,
            pltpu.SemaphoreType.DMA,
        ],
    )
    def k(x4_hbm, tok_hbm, out_hbm, idx_all, rows_v, pout, pos_v,
          gsem0, gsem1, gsem2, gsem3, ssem0, ssem1, ssem2, ssem3, isem):
        gsem = (gsem0, gsem1, gsem2, gsem3)
        ssem = (ssem0, ssem1, ssem2, ssem3)
        wid = lax.axis_index("s") * info.num_cores + lax.axis_index("c")
        first = wid * per_w

        idescs = []
        for sl in range(per_w):
            s = first + sl
            l = s >> 3
            idescs.append(pltpu.make_async_copy(
                x4_hbm.at[l >> 3, s & 7, l & 7], idx_all.at[sl], isem))
        for d in idescs:
            d.start()
        lbase = lax.min(first >> 3, _L - 8)
        pltpu.sync_copy(tok_hbm.at[pl.ds(vocab + lbase, 8)], pos_v)
        for d in idescs:
            d.wait()

        io = lax.iota(jnp.int32, 16)
        dt_idx = [(16 * kk + io) >> 3 for kk in range(_NVEC)]
        di_idx = [(16 * kk + io) & 7 for kk in range(_NVEC)]
        col_idx = [16 * kk + io for kk in range(_NVEC)]

        def gather_desc(buf, sl):
            return pltpu.make_async_copy(
                tok_hbm.at[idx_all.at[sl]], rows_v.at[buf], gsem[buf])

        def out_desc(buf, s):
            l = s >> 3
            bt = s & 7
            return pltpu.make_async_copy(
                pout.at[buf, :, :, pl.ds(0, 128)],
                out_hbm.at[l, :, bt], ssem[buf])

        def compute(buf, s):
            l = s >> 3
            pv = [pos_v[l - lbase, pl.ds(16 * kk, 16)] for kk in range(_NVEC)]

            @plsc.parallel_loop(0, 16, unroll=2)
            def _(r):
                rot = (io + r) & 15
                for m in range(8):
                    bvec = rot + (m * 16)
                    for kk in range(_NVEC):
                        v = plsc.load_gather(
                            rows_v.at[buf], [bvec, col_idx[kk]])
                        v = v + pv[kk]
                        plsc.store_scatter(
                            pout.at[buf], [dt_idx[kk], di_idx[kk], bvec], v)

        for i in range(4):
            gather_desc(i, i).start()

        n_main = (per_w - 2) // 4

        def gbody(h, carry):
            for buf in range(4):
                sl = 4 * h + buf
                s = first + sl
                gather_desc(buf, sl).wait()

                @pl.when(h > 0)
                def _():
                    out_desc(buf, s).wait()

                compute(buf, s)
                out_desc(buf, s).start()

                @pl.when(sl + 4 < per_w)
                def _():
                    gather_desc(buf, sl + 4).start()
            return carry

        lax.fori_loop(0, n_main, gbody, 0)
        for buf in range(2):
            sl = 4 * n_main + buf
            s = first + sl
            gather_desc(buf, sl).wait()
            out_desc(buf, s).wait()
            compute(buf, s)
            out_desc(buf, s).start()
        out_desc(0, first + per_w - 2).wait()
        out_desc(1, first + per_w - 1).wait()
        out_desc(2, first + per_w - 4).wait()
        out_desc(3, first + per_w - 3).wait()

    return k


def kernel(x, token_table, pos_table):
    b, l = x.shape
    x4 = x.astype(jnp.int32).reshape(8, 128, l // 8, 8).transpose(2, 0, 3, 1)
    aug = jnp.pad(
        jnp.concatenate([token_table, pos_table], axis=0),
        ((0, 0), (0, 128 - _D)))
    p = _build(token_table.shape[0])(x4, aug)
    return p.transpose(0, 1, 3, 2, 4).reshape(l, _D, b).transpose(2, 0, 1)
